# baseline (speedup 1.0000x reference)
import functools
import math

import jax
import jax.numpy as jnp
from jax.experimental import pallas as pl
from jax.experimental.pallas import tpu as pltpu

HEADS = 12
DIM_HEAD = 64
NUM_BUCKETS = 32
MAX_DISTANCE = 128
MASK_VALUE = -3.4028234663852886e38
MEM_SLOTS_PAD = 64


def _qkv_kernel(x_ref, w_ref, o_ref):
    o_ref[0] = jnp.dot(x_ref[...], w_ref[0],
                       preferred_element_type=jnp.float32)


def _out_kernel(a_ref, w_ref, bias_ref, o_ref):
    hi = pl.program_id(1)
    part = jnp.dot(a_ref[0], w_ref[0], preferred_element_type=jnp.float32)

    @pl.when(hi == 0)
    def _():
        o_ref[...] = part + bias_ref[...]

    @pl.when(hi > 0)
    def _():
        o_ref[...] += part


def _bias_kernel(tab_ref, o_ref, *, bq):
    o_ref[...] = jnp.zeros_like(o_ref)
    w = 3 * bq // 2
    r = jax.lax.broadcasted_iota(jnp.int32, (bq, w), 0)
    cc = jax.lax.broadcasted_iota(jnp.int32, (bq, w), 1)
    delta = bq // 2 + r - cc
    npos = jnp.maximum(delta, 0)
    max_exact = NUM_BUCKETS // 2
    safe = jnp.maximum(npos, 1).astype(jnp.float32)
    val_large = max_exact + (
        jnp.log(safe * (1.0 / max_exact))
        * (max_exact / math.log(MAX_DISTANCE / max_exact))
    ).astype(jnp.int32)
    bucket = jnp.where(npos < max_exact, npos,
                       jnp.minimum(val_large, NUM_BUCKETS - 1))
    t31 = tab_ref[0, 0, NUM_BUCKETS - 1]
    adj = jnp.zeros((bq, w), jnp.float32)
    for t in range(NUM_BUCKETS - 1):
        adj = adj + jnp.where(bucket == t, tab_ref[0, 0, t] - t31, 0.0)
    adj = jnp.where(delta >= 0, adj, 0.0)
    o_ref[0, :, bq // 2:2 * bq] = adj


def _attn_kernel(q_ref, k_ref, v_ref, w_ref, km_ref, vm_ref, tab_ref,
                 mask_ref, gate_ref, o_ref, s_ref, *, bq, n, scale):
    qi = pl.program_id(1)
    q = q_ref[0]

    t31 = tab_ref[0, 0, NUM_BUCKETS - 1]
    s_ref[...] = jnp.full((bq, n), MASK_VALUE, jnp.float32)

    def score_chunk(ki, carry):
        kc = k_ref[0, pl.ds(ki * bq, bq), :]
        sc = jax.lax.dot_general(q, kc, (((1,), (1,)), ((), ())),
                                 preferred_element_type=jnp.float32)
        s_ref[:, pl.ds(ki * bq, bq)] = sc * scale + t31
        return carry

    jax.lax.fori_loop(0, qi + 1, score_chunk, 0)

    start = jnp.maximum(qi - 1, 0) * bq
    wstart = jnp.where(qi == 0, bq, 0)
    s_ref[:, pl.ds(start, 2 * bq)] += w_ref[0, :, pl.ds(wstart, 2 * bq)]

    rr = jax.lax.broadcasted_iota(jnp.int32, (bq, bq), 0)
    cc = jax.lax.broadcasted_iota(jnp.int32, (bq, bq), 1)
    diag = s_ref[:, pl.ds(qi * bq, bq)]
    s_ref[:, pl.ds(qi * bq, bq)] = jnp.where(rr >= cc, diag, MASK_VALUE)

    s = s_ref[...]
    m = jnp.max(s, axis=1, keepdims=True)
    l = jnp.sum(jnp.exp(s - m), axis=1, keepdims=True)
    s_ref[...] = jnp.exp(s - m)

    def pv_chunk(ki, acc):
        pc = s_ref[:, pl.ds(ki * bq, bq)]
        vc = v_ref[0, pl.ds(ki * bq, bq), :]
        return acc + jnp.dot(pc, vc, preferred_element_type=jnp.float32)

    acc = jax.lax.fori_loop(0, qi + 1, pv_chunk,
                            jnp.zeros((bq, DIM_HEAD), jnp.float32))
    local = acc / l

    km = km_ref[0]
    vm = vm_ref[0]
    sm = jax.lax.dot_general(q, km, (((1,), (1,)), ((), ())),
                             preferred_element_type=jnp.float32) * scale
    mm = jnp.max(sm, axis=1, keepdims=True)
    pm = jnp.exp(sm - mm) * mask_ref[0, :, 0:MEM_SLOTS_PAD]
    lm = jnp.sum(pm, axis=1, keepdims=True)
    mem = jnp.dot(pm, vm, preferred_element_type=jnp.float32) / lm

    g = jax.nn.sigmoid(gate_ref[0, :, 0:1])
    o_ref[0] = local * g + mem * (1.0 - g)


def kernel(x, k_mem, v_mem, mem_mask, Wq, Wkv, Wo, bo, null_k, null_v,
           gate_param, rel_bias_table):
    b, n, dim = x.shape
    h, d = HEADS, DIM_HEAD
    topk = k_mem.shape[2]
    scale = d ** -0.5
    rel_scale = d ** 0.5
    nc = 3 * h

    x2 = x.reshape(n, dim)
    w3 = (jnp.concatenate([Wq, Wkv], axis=1)
          .reshape(dim, nc, d).transpose(1, 0, 2))

    bn = 512
    qkv = pl.pallas_call(
        _qkv_kernel,
        grid=(n // bn, nc),
        in_specs=[
            pl.BlockSpec((bn, dim), lambda i, c: (i, 0)),
            pl.BlockSpec((1, dim, d), lambda i, c: (c, 0, 0)),
        ],
        out_specs=pl.BlockSpec((1, bn, d), lambda i, c: (c, i, 0)),
        out_shape=jax.ShapeDtypeStruct((nc, n, d), jnp.float32),
    )(x2, w3)

    km = jnp.concatenate([null_k, k_mem[0]], axis=1)
    vm = jnp.concatenate([null_v, v_mem[0]], axis=1)
    pad = MEM_SLOTS_PAD - (1 + topk)
    km = jnp.pad(km, ((0, 0), (0, pad), (0, 0)))
    vm = jnp.pad(vm, ((0, 0), (0, pad), (0, 0)))
    maskf = jnp.concatenate(
        [jnp.ones((h, 1), jnp.float32), mem_mask[0].astype(jnp.float32),
         jnp.zeros((h, pad), jnp.float32)], axis=1)
    maskp = jnp.pad(maskf, ((0, 0), (0, 128 - MEM_SLOTS_PAD)))
    maskp = maskp.reshape(h, 1, 128)
    tabp = jnp.pad(rel_bias_table.T * rel_scale,
                   ((0, 0), (0, 128 - NUM_BUCKETS))).reshape(h, 1, 128)
    gatep = jnp.broadcast_to(gate_param.reshape(h, 1, 1),
                             (h, 1, 128)).astype(jnp.float32)

    bq = 256
    wtile = pl.pallas_call(
        functools.partial(_bias_kernel, bq=bq),
        grid=(h,),
        in_specs=[pl.BlockSpec((1, 1, 128), lambda hi: (hi, 0, 0))],
        out_specs=pl.BlockSpec((1, bq, 3 * bq), lambda hi: (hi, 0, 0)),
        out_shape=jax.ShapeDtypeStruct((h, bq, 3 * bq), jnp.float32),
    )(tabp)

    attn = pl.pallas_call(
        functools.partial(_attn_kernel, bq=bq, n=n, scale=scale),
        grid=(h, n // bq),
        in_specs=[
            pl.BlockSpec((1, bq, d), lambda hi, qi: (hi, qi, 0)),
            pl.BlockSpec((1, n, d), lambda hi, qi: (h + hi, 0, 0)),
            pl.BlockSpec((1, n, d), lambda hi, qi: (2 * h + hi, 0, 0)),
            pl.BlockSpec((1, bq, 3 * bq), lambda hi, qi: (hi, 0, 0)),
            pl.BlockSpec((1, MEM_SLOTS_PAD, d), lambda hi, qi: (hi, 0, 0)),
            pl.BlockSpec((1, MEM_SLOTS_PAD, d), lambda hi, qi: (hi, 0, 0)),
            pl.BlockSpec((1, 1, 128), lambda hi, qi: (hi, 0, 0)),
            pl.BlockSpec((1, 1, 128), lambda hi, qi: (hi, 0, 0)),
            pl.BlockSpec((1, 1, 128), lambda hi, qi: (hi, 0, 0)),
        ],
        out_specs=pl.BlockSpec((1, bq, d), lambda hi, qi: (hi, qi, 0)),
        out_shape=jax.ShapeDtypeStruct((h, n, d), jnp.float32),
        scratch_shapes=[pltpu.VMEM((bq, n), jnp.float32)],
    )(qkv, qkv, qkv, wtile, km, vm, tabp, maskp, gatep)

    wo3 = Wo.reshape(h, d, dim)
    bo2 = bo.reshape(1, dim)
    bn2 = 512
    out = pl.pallas_call(
        _out_kernel,
        grid=(n // bn2, h),
        in_specs=[
            pl.BlockSpec((1, bn2, d), lambda i, hi: (hi, i, 0)),
            pl.BlockSpec((1, d, dim), lambda i, hi: (hi, 0, 0)),
            pl.BlockSpec((1, dim), lambda i, hi: (0, 0)),
        ],
        out_specs=pl.BlockSpec((bn2, dim), lambda i, hi: (i, 0)),
        out_shape=jax.ShapeDtypeStruct((n, dim), jnp.float32),
    )(attn, wo3, bo2)

    return out.reshape(b, n, dim)


# speedup vs baseline: 1.6773x; 1.6773x over previous
import functools
import math

import jax
import jax.numpy as jnp
from jax.experimental import pallas as pl
from jax.experimental.pallas import tpu as pltpu

HEADS = 12
DIM_HEAD = 64
NUM_BUCKETS = 32
MAX_DISTANCE = 128
MASK_VALUE = -3.4028234663852886e38
MEM_SLOTS_PAD = 64


def _qkv_kernel(x_ref, w_ref, o_ref):
    o_ref[0] = jnp.dot(x_ref[...], w_ref[0],
                       preferred_element_type=jnp.float32
                       ).astype(jnp.bfloat16)


def _out_kernel(a_ref, w_ref, bias_ref, o_ref):
    hi = pl.program_id(1)
    part = jnp.dot(a_ref[0], w_ref[0], preferred_element_type=jnp.float32)

    @pl.when(hi == 0)
    def _():
        o_ref[...] = part + bias_ref[...]

    @pl.when(hi > 0)
    def _():
        o_ref[...] += part


def _bias_kernel(tab_ref, o_ref, *, bq):
    o_ref[...] = jnp.zeros_like(o_ref)
    w = 3 * bq // 2
    r = jax.lax.broadcasted_iota(jnp.int32, (bq, w), 0)
    cc = jax.lax.broadcasted_iota(jnp.int32, (bq, w), 1)
    delta = bq // 2 + r - cc
    npos = jnp.maximum(delta, 0)
    max_exact = NUM_BUCKETS // 2
    safe = jnp.maximum(npos, 1).astype(jnp.float32)
    val_large = max_exact + (
        jnp.log(safe * (1.0 / max_exact))
        * (max_exact / math.log(MAX_DISTANCE / max_exact))
    ).astype(jnp.int32)
    bucket = jnp.where(npos < max_exact, npos,
                       jnp.minimum(val_large, NUM_BUCKETS - 1))
    t31 = tab_ref[0, 0, NUM_BUCKETS - 1]
    adj = jnp.zeros((bq, w), jnp.float32)
    for t in range(NUM_BUCKETS - 1):
        adj = adj + jnp.where(bucket == t, tab_ref[0, 0, t] - t31, 0.0)
    adj = jnp.where(delta >= 0, adj, 0.0)
    o_ref[0, :, bq // 2:2 * bq] = adj


def _attn_kernel(q_ref, k_ref, v_ref, w_ref, km_ref, vm_ref, tab_ref,
                 mask_ref, gate_ref, o_ref, s_ref, *, bq, n, scale):
    qi = pl.program_id(1)
    q = q_ref[0]
    k = k_ref[0]
    v = v_ref[0]

    t31 = tab_ref[0, 0, NUM_BUCKETS - 1]
    s = jax.lax.dot_general(q, k, (((1,), (1,)), ((), ())),
                            preferred_element_type=jnp.float32) * scale + t31

    start = jnp.maximum(qi - 1, 0) * bq
    wstart = jnp.where(qi == 0, bq, 0)
    s_ref[...] = s
    s_ref[:, pl.ds(start, 2 * bq)] += w_ref[0, :, pl.ds(wstart, 2 * bq)]
    s = s_ref[...]

    i = qi * bq + jax.lax.broadcasted_iota(jnp.int32, (bq, n), 0)
    j = jax.lax.broadcasted_iota(jnp.int32, (bq, n), 1)
    s = jnp.where(j <= i, s, MASK_VALUE)

    m = jnp.max(s, axis=1, keepdims=True)
    p = jnp.exp(s - m)
    l = jnp.sum(p, axis=1, keepdims=True)
    local = jnp.dot(p.astype(jnp.bfloat16), v,
                    preferred_element_type=jnp.float32) / l

    km = km_ref[0]
    vm = vm_ref[0]
    sm = jax.lax.dot_general(q, km, (((1,), (1,)), ((), ())),
                             preferred_element_type=jnp.float32) * scale
    mm = jnp.max(sm, axis=1, keepdims=True)
    pm = jnp.exp(sm - mm) * mask_ref[0, :, 0:MEM_SLOTS_PAD]
    lm = jnp.sum(pm, axis=1, keepdims=True)
    mem = jnp.dot(pm.astype(jnp.bfloat16), vm,
                  preferred_element_type=jnp.float32) / lm

    g = jax.nn.sigmoid(gate_ref[0, :, 0:1])
    o_ref[0] = (local * g + mem * (1.0 - g)).astype(jnp.bfloat16)


def kernel(x, k_mem, v_mem, mem_mask, Wq, Wkv, Wo, bo, null_k, null_v,
           gate_param, rel_bias_table):
    b, n, dim = x.shape
    h, d = HEADS, DIM_HEAD
    topk = k_mem.shape[2]
    scale = d ** -0.5
    rel_scale = d ** 0.5
    nc = 3 * h

    x2 = x.reshape(n, dim).astype(jnp.bfloat16)
    w3 = (jnp.concatenate([Wq, Wkv], axis=1)
          .reshape(dim, nc, d).transpose(1, 0, 2)
          .astype(jnp.bfloat16))

    bn = 1024
    qkv = pl.pallas_call(
        _qkv_kernel,
        grid=(n // bn, nc),
        in_specs=[
            pl.BlockSpec((bn, dim), lambda i, c: (i, 0)),
            pl.BlockSpec((1, dim, d), lambda i, c: (c, 0, 0)),
        ],
        out_specs=pl.BlockSpec((1, bn, d), lambda i, c: (c, i, 0)),
        out_shape=jax.ShapeDtypeStruct((nc, n, d), jnp.bfloat16),
    )(x2, w3)

    km = jnp.concatenate([null_k, k_mem[0]], axis=1)
    vm = jnp.concatenate([null_v, v_mem[0]], axis=1)
    pad = MEM_SLOTS_PAD - (1 + topk)
    km = jnp.pad(km, ((0, 0), (0, pad), (0, 0))).astype(jnp.bfloat16)
    vm = jnp.pad(vm, ((0, 0), (0, pad), (0, 0))).astype(jnp.bfloat16)
    maskf = jnp.concatenate(
        [jnp.ones((h, 1), jnp.float32), mem_mask[0].astype(jnp.float32),
         jnp.zeros((h, pad), jnp.float32)], axis=1)
    maskp = jnp.pad(maskf, ((0, 0), (0, 128 - MEM_SLOTS_PAD)))
    maskp = maskp.reshape(h, 1, 128)
    tabp = jnp.pad(rel_bias_table.T * rel_scale,
                   ((0, 0), (0, 128 - NUM_BUCKETS))).reshape(h, 1, 128)
    gatep = jnp.broadcast_to(gate_param.reshape(h, 1, 1),
                             (h, 1, 128)).astype(jnp.float32)

    bq = 256
    wtile = pl.pallas_call(
        functools.partial(_bias_kernel, bq=bq),
        grid=(h,),
        in_specs=[pl.BlockSpec((1, 1, 128), lambda hi: (hi, 0, 0))],
        out_specs=pl.BlockSpec((1, bq, 3 * bq), lambda hi: (hi, 0, 0)),
        out_shape=jax.ShapeDtypeStruct((h, bq, 3 * bq), jnp.float32),
    )(tabp)

    attn = pl.pallas_call(
        functools.partial(_attn_kernel, bq=bq, n=n, scale=scale),
        grid=(h, n // bq),
        in_specs=[
            pl.BlockSpec((1, bq, d), lambda hi, qi: (hi, qi, 0)),
            pl.BlockSpec((1, n, d), lambda hi, qi: (h + hi, 0, 0)),
            pl.BlockSpec((1, n, d), lambda hi, qi: (2 * h + hi, 0, 0)),
            pl.BlockSpec((1, bq, 3 * bq), lambda hi, qi: (hi, 0, 0)),
            pl.BlockSpec((1, MEM_SLOTS_PAD, d), lambda hi, qi: (hi, 0, 0)),
            pl.BlockSpec((1, MEM_SLOTS_PAD, d), lambda hi, qi: (hi, 0, 0)),
            pl.BlockSpec((1, 1, 128), lambda hi, qi: (hi, 0, 0)),
            pl.BlockSpec((1, 1, 128), lambda hi, qi: (hi, 0, 0)),
            pl.BlockSpec((1, 1, 128), lambda hi, qi: (hi, 0, 0)),
        ],
        out_specs=pl.BlockSpec((1, bq, d), lambda hi, qi: (hi, qi, 0)),
        out_shape=jax.ShapeDtypeStruct((h, n, d), jnp.bfloat16),
        scratch_shapes=[pltpu.VMEM((bq, n), jnp.float32)],
    )(qkv, qkv, qkv, wtile, km, vm, tabp, maskp, gatep)

    wo3 = Wo.reshape(h, d, dim).astype(jnp.bfloat16)
    bo2 = bo.reshape(1, dim)
    bn2 = 512
    out = pl.pallas_call(
        _out_kernel,
        grid=(n // bn2, h),
        in_specs=[
            pl.BlockSpec((1, bn2, d), lambda i, hi: (hi, i, 0)),
            pl.BlockSpec((1, d, dim), lambda i, hi: (hi, 0, 0)),
            pl.BlockSpec((1, dim), lambda i, hi: (0, 0)),
        ],
        out_specs=pl.BlockSpec((bn2, dim), lambda i, hi: (i, 0)),
        out_shape=jax.ShapeDtypeStruct((n, dim), jnp.float32),
    )(attn, wo3, bo2)

    return out.reshape(b, n, dim)


# speedup vs baseline: 1.9910x; 1.1870x over previous
import functools
import math

import jax
import jax.numpy as jnp
from jax.experimental import pallas as pl
from jax.experimental.pallas import tpu as pltpu

HEADS = 12
DIM_HEAD = 64
NUM_BUCKETS = 32
MAX_DISTANCE = 128
MASK_VALUE = -3.4028234663852886e38
MEM_SLOTS_PAD = 64


def _qkv_kernel(x_ref, w_ref, o_ref):
    o_ref[0] = jnp.dot(x_ref[...], w_ref[0],
                       preferred_element_type=jnp.float32
                       ).astype(jnp.bfloat16)


def _out_kernel(a_ref, w_ref, bias_ref, o_ref):
    hi = pl.program_id(0)
    part = jnp.dot(a_ref[0], w_ref[0], preferred_element_type=jnp.float32)

    @pl.when(hi == 0)
    def _():
        o_ref[...] = part + bias_ref[...]

    @pl.when(hi > 0)
    def _():
        o_ref[...] += part


def _bias_kernel(tab_ref, o_ref, *, bq):
    o_ref[...] = jnp.zeros_like(o_ref)
    sr = 128
    w = 3 * sr
    r = jax.lax.broadcasted_iota(jnp.int32, (sr, w), 0)
    cc = jax.lax.broadcasted_iota(jnp.int32, (sr, w), 1)
    delta = 2 * sr + r - cc
    npos = jnp.maximum(delta, 0)
    max_exact = NUM_BUCKETS // 2
    safe = jnp.maximum(npos, 1).astype(jnp.float32)
    val_large = max_exact + (
        jnp.log(safe * (1.0 / max_exact))
        * (max_exact / math.log(MAX_DISTANCE / max_exact))
    ).astype(jnp.int32)
    bucket = jnp.where(npos < max_exact, npos,
                       jnp.minimum(val_large, NUM_BUCKETS - 1))
    t31 = tab_ref[0, 0, NUM_BUCKETS - 1]
    adj = jnp.zeros((sr, w), jnp.float32)
    for t in range(NUM_BUCKETS - 1):
        adj = adj + jnp.where(bucket == t, tab_ref[0, 0, t] - t31, 0.0)
    adj = jnp.where(delta >= 0, adj, 0.0)
    for ri in range(bq // sr):
        cs = bq + (ri - 2) * sr
        o_ref[0, ri * sr:(ri + 1) * sr, cs:cs + w] = adj


def _attn_kernel(q_ref, k_ref, v_ref, w_ref, km_ref, vm_ref, tab_ref,
                 mask_ref, gate_ref, o_ref, s_ref, *, bq, n, scale):
    qi = pl.program_id(1)
    q = q_ref[0]
    k = k_ref[0]
    v = v_ref[0]

    t31 = tab_ref[0, 0, NUM_BUCKETS - 1]
    s = jax.lax.dot_general(q, k, (((1,), (1,)), ((), ())),
                            preferred_element_type=jnp.float32) * scale + t31

    start = jnp.maximum(qi - 1, 0) * bq
    wstart = jnp.where(qi == 0, bq, 0)
    s_ref[...] = s
    s_ref[:, pl.ds(start, 2 * bq)] += w_ref[0, :, pl.ds(wstart, 2 * bq)]
    s = s_ref[...]

    i = qi * bq + jax.lax.broadcasted_iota(jnp.int32, (bq, n), 0)
    j = jax.lax.broadcasted_iota(jnp.int32, (bq, n), 1)
    s = jnp.where(j <= i, s, MASK_VALUE)

    m = jnp.max(s, axis=1, keepdims=True)
    p = jnp.exp(s - m)
    l = jnp.sum(p, axis=1, keepdims=True)
    local = jnp.dot(p.astype(jnp.bfloat16), v,
                    preferred_element_type=jnp.float32) / l

    km = km_ref[0]
    vm = vm_ref[0]
    sm = jax.lax.dot_general(q, km, (((1,), (1,)), ((), ())),
                             preferred_element_type=jnp.float32) * scale
    mm = jnp.max(sm, axis=1, keepdims=True)
    pm = jnp.exp(sm - mm) * mask_ref[0, :, 0:MEM_SLOTS_PAD]
    lm = jnp.sum(pm, axis=1, keepdims=True)
    mem = jnp.dot(pm.astype(jnp.bfloat16), vm,
                  preferred_element_type=jnp.float32) / lm

    g = jax.nn.sigmoid(gate_ref[0, :, 0:1])
    o_ref[0] = (local * g + mem * (1.0 - g)).astype(jnp.bfloat16)


def kernel(x, k_mem, v_mem, mem_mask, Wq, Wkv, Wo, bo, null_k, null_v,
           gate_param, rel_bias_table):
    b, n, dim = x.shape
    h, d = HEADS, DIM_HEAD
    topk = k_mem.shape[2]
    scale = d ** -0.5
    rel_scale = d ** 0.5
    nc = 3 * h

    x2 = x.reshape(n, dim).astype(jnp.bfloat16)
    w3 = (jnp.concatenate([Wq, Wkv], axis=1)
          .reshape(dim, nc, d).transpose(1, 0, 2)
          .astype(jnp.bfloat16))

    qkv = pl.pallas_call(
        _qkv_kernel,
        grid=(nc,),
        in_specs=[
            pl.BlockSpec((n, dim), lambda c: (0, 0)),
            pl.BlockSpec((1, dim, d), lambda c: (c, 0, 0)),
        ],
        out_specs=pl.BlockSpec((1, n, d), lambda c: (c, 0, 0)),
        out_shape=jax.ShapeDtypeStruct((nc, n, d), jnp.bfloat16),
    )(x2, w3)

    km = jnp.concatenate([null_k, k_mem[0]], axis=1)
    vm = jnp.concatenate([null_v, v_mem[0]], axis=1)
    pad = MEM_SLOTS_PAD - (1 + topk)
    km = jnp.pad(km, ((0, 0), (0, pad), (0, 0))).astype(jnp.bfloat16)
    vm = jnp.pad(vm, ((0, 0), (0, pad), (0, 0))).astype(jnp.bfloat16)
    maskf = jnp.concatenate(
        [jnp.ones((h, 1), jnp.float32), mem_mask[0].astype(jnp.float32),
         jnp.zeros((h, pad), jnp.float32)], axis=1)
    maskp = jnp.pad(maskf, ((0, 0), (0, 128 - MEM_SLOTS_PAD)))
    maskp = maskp.reshape(h, 1, 128)
    tabp = jnp.pad(rel_bias_table.T * rel_scale,
                   ((0, 0), (0, 128 - NUM_BUCKETS))).reshape(h, 1, 128)
    gatep = jnp.broadcast_to(gate_param.reshape(h, 1, 1),
                             (h, 1, 128)).astype(jnp.float32)

    bq = 256
    wtile = pl.pallas_call(
        functools.partial(_bias_kernel, bq=bq),
        grid=(h,),
        in_specs=[pl.BlockSpec((1, 1, 128), lambda hi: (hi, 0, 0))],
        out_specs=pl.BlockSpec((1, bq, 3 * bq), lambda hi: (hi, 0, 0)),
        out_shape=jax.ShapeDtypeStruct((h, bq, 3 * bq), jnp.float32),
    )(tabp)

    attn = pl.pallas_call(
        functools.partial(_attn_kernel, bq=bq, n=n, scale=scale),
        grid=(h, n // bq),
        in_specs=[
            pl.BlockSpec((1, bq, d), lambda hi, qi: (hi, qi, 0)),
            pl.BlockSpec((1, n, d), lambda hi, qi: (h + hi, 0, 0)),
            pl.BlockSpec((1, n, d), lambda hi, qi: (2 * h + hi, 0, 0)),
            pl.BlockSpec((1, bq, 3 * bq), lambda hi, qi: (hi, 0, 0)),
            pl.BlockSpec((1, MEM_SLOTS_PAD, d), lambda hi, qi: (hi, 0, 0)),
            pl.BlockSpec((1, MEM_SLOTS_PAD, d), lambda hi, qi: (hi, 0, 0)),
            pl.BlockSpec((1, 1, 128), lambda hi, qi: (hi, 0, 0)),
            pl.BlockSpec((1, 1, 128), lambda hi, qi: (hi, 0, 0)),
            pl.BlockSpec((1, 1, 128), lambda hi, qi: (hi, 0, 0)),
        ],
        out_specs=pl.BlockSpec((1, bq, d), lambda hi, qi: (hi, qi, 0)),
        out_shape=jax.ShapeDtypeStruct((h, n, d), jnp.bfloat16),
        scratch_shapes=[pltpu.VMEM((bq, n), jnp.float32)],
    )(qkv, qkv, qkv, wtile, km, vm, tabp, maskp, gatep)

    wo3 = Wo.reshape(h, d, dim).astype(jnp.bfloat16)
    bo2 = bo.reshape(1, dim)
    out = pl.pallas_call(
        _out_kernel,
        grid=(h,),
        in_specs=[
            pl.BlockSpec((1, n, d), lambda hi: (hi, 0, 0)),
            pl.BlockSpec((1, d, dim), lambda hi: (hi, 0, 0)),
            pl.BlockSpec((1, dim), lambda hi: (0, 0)),
        ],
        out_specs=pl.BlockSpec((n, dim), lambda hi: (0, 0)),
        out_shape=jax.ShapeDtypeStruct((n, dim), jnp.float32),
    )(attn, wo3, bo2)

    return out.reshape(b, n, dim)
